# baseline (speedup 1.0000x reference)
import jax
import jax.numpy as jnp
from jax.experimental import pallas as pl
from jax.experimental.pallas import tpu as pltpu


_BR = 256
_WGRP = 4


def _pack_kernel(a_ref, s_ref, a8_ref):
    a = a_ref[:, :]
    br = a.shape[0]
    sub = pl.program_id(0) % _WGRP
    ones = jnp.ones((a.shape[1], 128), dtype=jnp.bfloat16)
    acc = jax.lax.dot_general(
        a.astype(jnp.bfloat16), ones, (((1,), (0,)), ((), ())),
        preferred_element_type=jnp.float32)
    s_ref[pl.ds(sub * br, br), :] = jnp.sqrt(acc[:, :1])
    a8_ref[pl.ds(sub * br, br), :] = a.astype(jnp.int8)


def _z_kernel(s_ref, x_ref, w_ref, z_ref):
    z = jnp.dot(s_ref[:, :] * x_ref[:, :], w_ref[:, :],
                preferred_element_type=jnp.float32)
    z_ref[:, :] = z.astype(jnp.bfloat16)


def _spmm_kernel(z_ref, a8_ref, s_blk_ref, o_ref):
    acc = jax.lax.dot_general(
        a8_ref[:, :], z_ref[:, :], (((1,), (0,)), ((), ())),
        preferred_element_type=jnp.float32)
    o_ref[:, :] = s_blk_ref[:, :] * acc


def kernel(X, A, W):
    n, d = X.shape
    br = _BR
    nb = pl.cdiv(n, br)
    s, a8 = pl.pallas_call(
        _pack_kernel,
        grid=(nb,),
        in_specs=[pl.BlockSpec((br, n), lambda i: (i, 0))],
        out_specs=[
            pl.BlockSpec((br * _WGRP, 1), lambda i: (i // _WGRP, 0)),
            pl.BlockSpec((br * _WGRP, n), lambda i: (i // _WGRP, 0)),
        ],
        out_shape=[
            jax.ShapeDtypeStruct((n, 1), jnp.float32),
            jax.ShapeDtypeStruct((n, n), jnp.int8),
        ],
    )(A)

    z = pl.pallas_call(
        _z_kernel,
        in_specs=[
            pl.BlockSpec((n, 1), lambda: (0, 0)),
            pl.BlockSpec((n, d), lambda: (0, 0)),
            pl.BlockSpec((d, d), lambda: (0, 0)),
        ],
        out_specs=pl.BlockSpec((n, d), lambda: (0, 0)),
        out_shape=jax.ShapeDtypeStruct((n, d), jnp.bfloat16),
    )(s, X, W)

    out = pl.pallas_call(
        _spmm_kernel,
        grid=(nb,),
        in_specs=[
            pl.BlockSpec((n, d), lambda i: (0, 0)),
            pl.BlockSpec((br, n), lambda i: (i, 0)),
            pl.BlockSpec((br, 1), lambda i: (i, 0)),
        ],
        out_specs=pl.BlockSpec((br, d), lambda i: (i, 0)),
        out_shape=jax.ShapeDtypeStruct((n, d), jnp.float32),
    )(z, a8, s)

    return out


# speedup vs baseline: 1.0559x; 1.0559x over previous
import jax
import jax.numpy as jnp
from jax.experimental import pallas as pl
from jax.experimental.pallas import tpu as pltpu


_BR = 512


def _pack_kernel(a_ref, s_ref, a8_ref):
    a = a_ref[:, :]
    ones = jnp.ones((a.shape[1], 128), dtype=jnp.bfloat16)
    acc = jax.lax.dot_general(
        a.astype(jnp.bfloat16), ones, (((1,), (0,)), ((), ())),
        preferred_element_type=jnp.float32)
    s_ref[:, :] = jnp.sqrt(acc[:, :1])
    a8_ref[:, :] = a.astype(jnp.int8)


def _z_kernel(s_ref, x_ref, w_ref, z_ref):
    z = jnp.dot(s_ref[:, :] * x_ref[:, :], w_ref[:, :],
                preferred_element_type=jnp.float32)
    z_ref[:, :] = z.astype(jnp.bfloat16)


def _spmm_kernel(z_ref, a8_ref, s_blk_ref, o_ref):
    acc = jax.lax.dot_general(
        a8_ref[:, :], z_ref[:, :], (((1,), (0,)), ((), ())),
        preferred_element_type=jnp.float32)
    o_ref[:, :] = s_blk_ref[:, :] * acc


def kernel(X, A, W):
    n, d = X.shape
    br = _BR
    nb = pl.cdiv(n, br)
    br1 = 512
    nb1 = pl.cdiv(n, br1)

    s, a8 = pl.pallas_call(
        _pack_kernel,
        grid=(nb1,),
        in_specs=[pl.BlockSpec((br1, n), lambda i: (i, 0))],
        out_specs=[
            pl.BlockSpec((br1, 1), lambda i: (i, 0)),
            pl.BlockSpec((br1, n), lambda i: (i, 0)),
        ],
        out_shape=[
            jax.ShapeDtypeStruct((n, 1), jnp.float32),
            jax.ShapeDtypeStruct((n, n), jnp.int8),
        ],
    )(A)

    z = pl.pallas_call(
        _z_kernel,
        in_specs=[
            pl.BlockSpec((n, 1), lambda: (0, 0)),
            pl.BlockSpec((n, d), lambda: (0, 0)),
            pl.BlockSpec((d, d), lambda: (0, 0)),
        ],
        out_specs=pl.BlockSpec((n, d), lambda: (0, 0)),
        out_shape=jax.ShapeDtypeStruct((n, d), jnp.bfloat16),
    )(s, X, W)

    out = pl.pallas_call(
        _spmm_kernel,
        grid=(nb,),
        in_specs=[
            pl.BlockSpec((n, d), lambda i: (0, 0)),
            pl.BlockSpec((br, n), lambda i: (i, 0)),
            pl.BlockSpec((br, 1), lambda i: (i, 0)),
        ],
        out_specs=pl.BlockSpec((br, d), lambda i: (i, 0)),
        out_shape=jax.ShapeDtypeStruct((n, d), jnp.float32),
    )(z, a8, s)

    return out


# speedup vs baseline: 1.0584x; 1.0024x over previous
import jax
import jax.numpy as jnp
from jax.experimental import pallas as pl
from jax.experimental.pallas import tpu as pltpu


_BR = 1024


def _pack_kernel(a_ref, s_ref, a8_ref):
    a = a_ref[:, :]
    ones = jnp.ones((a.shape[1], 128), dtype=jnp.bfloat16)
    acc = jax.lax.dot_general(
        a.astype(jnp.bfloat16), ones, (((1,), (0,)), ((), ())),
        preferred_element_type=jnp.float32)
    s_ref[:, :] = jnp.sqrt(acc[:, :1])
    a8_ref[:, :] = a.astype(jnp.int8)


def _z_kernel(s_ref, x_ref, w_ref, z_ref):
    z = jnp.dot(s_ref[:, :] * x_ref[:, :], w_ref[:, :],
                preferred_element_type=jnp.float32)
    z_ref[:, :] = z.astype(jnp.bfloat16)


def _spmm_kernel(z_ref, a8_ref, s_blk_ref, o_ref):
    acc = jax.lax.dot_general(
        a8_ref[:, :], z_ref[:, :], (((1,), (0,)), ((), ())),
        preferred_element_type=jnp.float32)
    o_ref[:, :] = s_blk_ref[:, :] * acc


def kernel(X, A, W):
    n, d = X.shape
    br = _BR
    nb = pl.cdiv(n, br)
    br1 = 512
    nb1 = pl.cdiv(n, br1)

    s, a8 = pl.pallas_call(
        _pack_kernel,
        grid=(nb1,),
        in_specs=[pl.BlockSpec((br1, n), lambda i: (i, 0))],
        out_specs=[
            pl.BlockSpec((br1, 1), lambda i: (i, 0)),
            pl.BlockSpec((br1, n), lambda i: (i, 0)),
        ],
        out_shape=[
            jax.ShapeDtypeStruct((n, 1), jnp.float32),
            jax.ShapeDtypeStruct((n, n), jnp.int8),
        ],
    )(A)

    z = pl.pallas_call(
        _z_kernel,
        in_specs=[
            pl.BlockSpec((n, 1), lambda: (0, 0)),
            pl.BlockSpec((n, d), lambda: (0, 0)),
            pl.BlockSpec((d, d), lambda: (0, 0)),
        ],
        out_specs=pl.BlockSpec((n, d), lambda: (0, 0)),
        out_shape=jax.ShapeDtypeStruct((n, d), jnp.bfloat16),
    )(s, X, W)

    out = pl.pallas_call(
        _spmm_kernel,
        grid=(nb,),
        in_specs=[
            pl.BlockSpec((n, d), lambda i: (0, 0)),
            pl.BlockSpec((br, n), lambda i: (i, 0)),
            pl.BlockSpec((br, 1), lambda i: (i, 0)),
        ],
        out_specs=pl.BlockSpec((br, d), lambda i: (i, 0)),
        out_shape=jax.ShapeDtypeStruct((n, d), jnp.float32),
    )(z, a8, s)

    return out


# speedup vs baseline: 1.0770x; 1.0176x over previous
import jax
import jax.numpy as jnp
from jax.experimental import pallas as pl
from jax.experimental.pallas import tpu as pltpu


_BR = 1024


def _pack_kernel(a_ref, s_ref, a8_ref):
    a = a_ref[:, :]
    ones = jnp.ones((a.shape[1], 128), dtype=jnp.bfloat16)
    acc = jax.lax.dot_general(
        a.astype(jnp.bfloat16), ones, (((1,), (0,)), ((), ())),
        preferred_element_type=jnp.float32)
    s_ref[:, :] = jnp.sqrt(acc[:, :1])
    a8_ref[:, :] = a.astype(jnp.int8)


def _z_kernel(s_ref, x_ref, w_ref, z_ref):
    z = jnp.dot(s_ref[:, :] * x_ref[:, :], w_ref[:, :],
                preferred_element_type=jnp.float32)
    z_ref[:, :] = z.astype(jnp.bfloat16)


def _spmm_kernel(s_full_ref, x_ref, w_ref, a8_ref, s_blk_ref, o_ref, z_ref):
    @pl.when(pl.program_id(0) == 0)
    def _init_z():
        z = jnp.dot(s_full_ref[:, :] * x_ref[:, :], w_ref[:, :],
                    preferred_element_type=jnp.float32)
        z_ref[:, :] = z.astype(jnp.bfloat16)

    acc = jax.lax.dot_general(
        a8_ref[:, :], z_ref[:, :], (((1,), (0,)), ((), ())),
        preferred_element_type=jnp.float32)
    o_ref[:, :] = s_blk_ref[:, :] * acc


def kernel(X, A, W):
    n, d = X.shape
    br = _BR
    nb = pl.cdiv(n, br)
    br1 = 512
    nb1 = pl.cdiv(n, br1)

    s, a8 = pl.pallas_call(
        _pack_kernel,
        grid=(nb1,),
        in_specs=[pl.BlockSpec((br1, n), lambda i: (i, 0))],
        out_specs=[
            pl.BlockSpec((br1, 1), lambda i: (i, 0)),
            pl.BlockSpec((br1, n), lambda i: (i, 0)),
        ],
        out_shape=[
            jax.ShapeDtypeStruct((n, 1), jnp.float32),
            jax.ShapeDtypeStruct((n, n), jnp.int8),
        ],
    )(A)

    out = pl.pallas_call(
        _spmm_kernel,
        grid=(nb,),
        in_specs=[
            pl.BlockSpec((n, 1), lambda i: (0, 0)),
            pl.BlockSpec((n, d), lambda i: (0, 0)),
            pl.BlockSpec((d, d), lambda i: (0, 0)),
            pl.BlockSpec((br, n), lambda i: (i, 0)),
            pl.BlockSpec((br, 1), lambda i: (i, 0)),
        ],
        out_specs=pl.BlockSpec((br, d), lambda i: (i, 0)),
        out_shape=jax.ShapeDtypeStruct((n, d), jnp.float32),
        scratch_shapes=[pltpu.VMEM((n, d), jnp.bfloat16)],
    )(s, X, W, a8, s)

    return out
